# baseline (speedup 1.0000x reference)
import functools

import jax
import jax.numpy as jnp
from jax import lax
from jax.experimental import pallas as pl
from jax.experimental.pallas import tpu as pltpu
from jax.experimental.pallas import tpu_sc as plsc

N = 10000
E = 160000
D_IN = 128
D = 1024
H = 4
DH = 256
D_FF = 4096
OUT = 256
L = 4

NPAD = 10240
RB = 256
NBLK = NPAD // RB

NTILE = 32
EPAD = 163840
ET = EPAD // NTILE
SB = 128
NSB = ET // SB
NC = 8
NF = NPAD * H
NQ = 4
NF4 = NF // NQ
NFS = NF4 // 16


def _shuf16(vec, perm):
    return lax.gather(
        vec, perm.reshape(16, 1),
        dimension_numbers=lax.GatherDimensionNumbers(
            offset_dims=(), collapsed_slice_dims=(0,), start_index_map=(0,)),
        slice_sizes=(1,),
        mode=lax.GatherScatterMode.PROMISE_IN_BOUNDS)


def _hsum16(vec):
    iota = lax.iota(jnp.int32, 16)
    for s in (8, 4, 2, 1):
        vec = vec + _shuf16(vec, iota ^ s)
    return vec


def _bc16(vec, i):
    idx = jnp.full((16, 1), i, jnp.int32)
    return lax.gather(
        vec, idx,
        dimension_numbers=lax.GatherDimensionNumbers(
            offset_dims=(), collapsed_slice_dims=(0,), start_index_map=(0,)),
        slice_sizes=(1,),
        mode=lax.GatherScatterMode.PROMISE_IN_BOUNDS)


_IOTA = None


SBA = 64
NSBA = ET // SBA


def _fire(z_refs, c0, c1, sidx, didx, bufs, sem):
    pltpu.async_copy(z_refs[c0].at[sidx], bufs[0], sem)
    pltpu.async_copy(z_refs[c1].at[sidx], bufs[1], sem)
    pltpu.async_copy(z_refs[c0].at[didx], bufs[2], sem)
    pltpu.async_copy(z_refs[c1].at[didx], bufs[3], sem)


def _drain(z_refs, bufs, sem):
    for b in bufs:
        pltpu.make_async_copy(z_refs[0], b, sem).wait()


def _edge_logits_body(z_refs, a_ref, src_ref, dst_ref, e_out, mxp_out,
                      a_v, b0, b1, b2, b3, b4, b5, b6, b7,
                      sidx0, didx0, sidx1, didx1, e_sb, mx, acc, tmp,
                      shared, sem0, sem1):
    cid = lax.axis_index("c")
    sid = lax.axis_index("s")
    wid = cid * 16 + sid
    base = wid * ET
    iota = lax.iota(jnp.int32, 16)
    lane15 = iota == 15
    hd = iota & 3
    msk = iota < 4
    sets = [[b0, b1, b2, b3], [b4, b5, b6, b7]]
    idxsets = [[sidx0, didx0], [sidx1, didx1]]
    sems = [sem0, sem1]

    pltpu.sync_copy(a_ref, a_v)

    def init_mx(v, _):
        mx[pl.ds(v * 16, 16)] = jnp.full((16,), -1e30, jnp.float32)
        return 0
    lax.fori_loop(0, NF // 16, init_mx, 0)

    a_all = [[a_v[c, pl.ds(dv * 16, 16)] for dv in range(8)]
             for c in range(NC)]

    def compute_hp(hp, bufs, didx):
        zsA, zsB, zdA, zdB = bufs
        aA = a_all[2 * hp]
        aB = a_all[2 * hp + 1]

        def edge(j, _):
            accv = jnp.zeros((16,), jnp.float32)
            for dv in range(8):
                sl = pl.ds(dv * 16, 16)
                t0 = zsA[j, sl] + zdA[j, sl]
                t1 = zsB[j, sl] + zdB[j, sl]
                accv += jnp.maximum(t0, t0 * 0.2) * aA[dv]
                accv += jnp.maximum(t1, t1 * 0.2) * aB[dv]
            tot = _hsum16(accv)
            plsc.store_scatter(
                e_sb, [jnp.full((16,), hp * SBA + j, jnp.int32)], tot,
                mask=lane15)
            return 0
        lax.fori_loop(0, SBA, edge, 0)

    def mx_pass(didx):
        def upd(j, _):
            jj = jnp.full((16,), j, jnp.int32)
            dstb = plsc.load_gather(didx, [jj])
            ev = plsc.load_gather(e_sb, [hd * SBA + jj], mask=msk)
            idxv = hd * NPAD + dstb
            cur = plsc.load_gather(mx, [idxv], mask=msk)
            plsc.store_scatter(mx, [idxv], jnp.maximum(cur, ev), mask=msk)
            return 0
        lax.fori_loop(0, SBA, upd, 0)

    def write_e(eb):
        for hh in range(H):
            pltpu.sync_copy(e_sb.at[pl.ds(hh * SBA, SBA)],
                            e_out.at[hh, pl.ds(eb, SBA)])

    pltpu.sync_copy(src_ref.at[pl.ds(base, SBA)], sidx0)
    pltpu.sync_copy(dst_ref.at[pl.ds(base, SBA)], didx0)
    _fire(z_refs, 0, 1, sidx0, didx0, sets[0], sems[0])

    def pair(k, _):
        for p in range(2):
            sb = k * 2 + p
            eb = base + sb * SBA
            s_idx, d_idx = idxsets[p]
            n_idx = idxsets[1 - p]
            for hp in range(H):
                if hp < 3:
                    _fire(z_refs, 2 * hp + 2, 2 * hp + 3, s_idx, d_idx,
                          sets[(hp + 1) & 1], sems[(hp + 1) & 1])
                else:
                    nb = eb + SBA if p == 0 else base + (k + 1) * 2 * SBA

                    @pl.when(nb < base + ET)
                    def _():
                        pltpu.sync_copy(src_ref.at[pl.ds(nb, SBA)], n_idx[0])
                        pltpu.sync_copy(dst_ref.at[pl.ds(nb, SBA)], n_idx[1])
                        _fire(z_refs, 0, 1, n_idx[0], n_idx[1], sets[0],
                              sems[0])
                _drain(z_refs, sets[hp & 1], sems[hp & 1])
                compute_hp(hp, sets[hp & 1], d_idx)
            write_e(eb)
            mx_pass(d_idx)
        return 0

    lax.fori_loop(0, NSBA // 2, pair, 0)

    for q in range(NQ):
        plsc.subcore_barrier()
        pltpu.sync_copy(mx.at[pl.ds(q * NF4, NF4)], shared.at[sid])
        plsc.subcore_barrier()
        off = sid * NFS
        pltpu.sync_copy(shared.at[0, pl.ds(off, NFS)], acc)
        for t in range(1, 16):
            pltpu.sync_copy(shared.at[t, pl.ds(off, NFS)], tmp)

            def mx_comb(v, _):
                sl = pl.ds(v * 16, 16)
                acc[sl] = jnp.maximum(acc[sl], tmp[sl])
                return 0
            lax.fori_loop(0, NFS // 16, mx_comb, 0)
        pltpu.sync_copy(acc, mxp_out.at[cid, pl.ds(q * NF4 + off, NFS)])


def _edge_logits(z_chunks, a8, srcP, dstP):
    mesh = plsc.VectorSubcoreMesh(core_axis_name="c", subcore_axis_name="s")
    body = functools.partial(
        pl.kernel,
        out_type=[
            jax.ShapeDtypeStruct((H, EPAD), jnp.float32),
            jax.ShapeDtypeStruct((2, NF), jnp.float32),
        ],
        mesh=mesh,
        scratch_types=[
            pltpu.VMEM((NC, 128), jnp.float32),
        ] + [pltpu.VMEM((SBA, 128), jnp.float32) for _ in range(8)] + [
            pltpu.VMEM((SBA,), jnp.int32),
            pltpu.VMEM((SBA,), jnp.int32),
            pltpu.VMEM((SBA,), jnp.int32),
            pltpu.VMEM((SBA,), jnp.int32),
            pltpu.VMEM((H * SBA,), jnp.float32),
            pltpu.VMEM((NF,), jnp.float32),
            pltpu.VMEM((NFS,), jnp.float32),
            pltpu.VMEM((NFS,), jnp.float32),
            pltpu.VMEM_SHARED((16, NF4), jnp.float32),
            pltpu.SemaphoreType.DMA,
            pltpu.SemaphoreType.DMA,
        ],
        compiler_params=pltpu.CompilerParams(needs_layout_passes=False),
    )

    def wrapped(*refs):
        z_refs = refs[:NC]
        rest = refs[NC:]
        _edge_logits_body(z_refs, *rest)

    return body(wrapped)(*z_chunks, a8, srcP, dstP)


def _ln(x, g, b):
    m = jnp.mean(x, axis=-1, keepdims=True)
    v = jnp.mean(jnp.square(x - m), axis=-1, keepdims=True)
    return (x - m) * jax.lax.rsqrt(v + 1e-3) * g + b


def _head_body(x_ref, w_ref, b_ref, o_ref):
    o_ref[...] = jnp.dot(x_ref[...], w_ref[...],
                         preferred_element_type=jnp.float32) + b_ref[...]


def _head(x, W0, b0):
    return pl.pallas_call(
        _head_body,
        grid=(NBLK,),
        in_specs=[
            pl.BlockSpec((RB, D_IN), lambda i: (i, 0)),
            pl.BlockSpec((D_IN, D), lambda i: (0, 0)),
            pl.BlockSpec((1, D), lambda i: (0, 0)),
        ],
        out_specs=pl.BlockSpec((RB, D), lambda i: (i, 0)),
        out_shape=jax.ShapeDtypeStruct((NPAD, D), jnp.float32),
    )(x, W0, b0.reshape(1, D))


EPT = EPAD // 16
NSB2 = EPT // SB
NROWT = NPAD // 16
NV = SB // 16


def _agg_body(z_refs, src_ref, dst_ref, e_ref, mxp_ref, agg_refs,
              zbuf, sidx, didx, didx2, ebuf, mxh, denh, exb, albuf,
              zrows, msgbuf, t0, t1, mxg, deng, aggsh, sem):
    cid = lax.axis_index("c")
    sid = lax.axis_index("s")

    def zero_zbuf(v, _):
        zbuf[v // 8, pl.ds((v % 8) * 16, 16)] = jnp.zeros((16,), jnp.float32)
        return 0
    lax.fori_loop(0, 128, zero_zbuf, 0)

    off = sid * (NF // 16)
    pltpu.sync_copy(mxp_ref.at[0, pl.ds(off, NF // 16)], t0)
    pltpu.sync_copy(mxp_ref.at[1, pl.ds(off, NF // 16)], t1)

    def mx_comb(v, _):
        sl = pl.ds(v * 16, 16)
        t0[sl] = jnp.maximum(t0[sl], t1[sl])
        return 0
    lax.fori_loop(0, NF // 256, mx_comb, 0)
    pltpu.sync_copy(t0, mxg.at[pl.ds(off, NF // 16)])

    def zt1(v, _):
        t1[pl.ds(v * 16, 16)] = jnp.zeros((16,), jnp.float32)
        return 0
    lax.fori_loop(0, NF // 256, zt1, 0)
    pltpu.sync_copy(t1, deng.at[pl.ds(off, NF // 16)])
    plsc.subcore_barrier()

    def sb_den(sb, _):
        eb = sid * EPT + sb * SB
        pltpu.sync_copy(dst_ref.at[pl.ds(eb, SB)], didx)
        for h in range(H):
            pltpu.sync_copy(e_ref.at[h, pl.ds(eb, SB)], ebuf)

            def mkidx(v, _):
                sl = pl.ds(v * 16, 16)
                didx2[sl] = didx[sl] + h * NPAD
                return 0
            lax.fori_loop(0, NV, mkidx, 0)
            pltpu.async_copy(mxg.at[didx2], mxh, sem).wait()

            def exv(v, _):
                sl = pl.ds(v * 16, 16)
                exb[sl] = jnp.exp(ebuf[sl] - mxh[sl])
                return 0
            lax.fori_loop(0, NV, exv, 0)
            pltpu.sync_copy(exb, deng.at[didx2], add=True)
        return 0
    lax.fori_loop(0, NSB2, sb_den, 0)
    plsc.subcore_barrier()

    for c in range(NC):
        @pl.when(cid == c // 4)
        def _():
            def zrow(r, _):
                pltpu.sync_copy(zbuf, aggsh.at[pl.ds(sid * NROWT + r * 16, 16), :])
                return 0
            lax.fori_loop(0, NROWT // 16, zrow, 0)
        plsc.subcore_barrier()

        @pl.when(cid == c // 4)
        def _():
            h = c >> 1

            def superbatch(sb, _):
                eb = sid * EPT + sb * SB
                pltpu.sync_copy(src_ref.at[pl.ds(eb, SB)], sidx)
                pltpu.sync_copy(dst_ref.at[pl.ds(eb, SB)], didx)
                pltpu.sync_copy(e_ref.at[h, pl.ds(eb, SB)], ebuf)

                def mkidx(v, _):
                    sl = pl.ds(v * 16, 16)
                    didx2[sl] = didx[sl] + h * NPAD
                    return 0
                lax.fori_loop(0, NV, mkidx, 0)
                pltpu.async_copy(mxg.at[didx2], mxh, sem).wait()
                pltpu.async_copy(deng.at[didx2], denh, sem).wait()
                pltpu.async_copy(z_refs[c].at[sidx], zrows, sem).wait()

                def alv(v, _):
                    sl = pl.ds(v * 16, 16)
                    albuf[sl] = jnp.exp(ebuf[sl] - mxh[sl]) / (denh[sl] + 1e-9)
                    return 0
                lax.fori_loop(0, NV, alv, 0)

                def edge(j, _):
                    jj = jnp.full((16,), j, jnp.int32)
                    albc = plsc.load_gather(albuf, [jj])
                    for dvv in range(8):
                        sl = pl.ds(dvv * 16, 16)
                        msgbuf[j, sl] = zrows[j, sl] * albc
                    return 0
                lax.fori_loop(0, SB, edge, 0)
                pltpu.sync_copy(msgbuf, aggsh.at[didx], add=True)
                return 0

            lax.fori_loop(0, NSB2, superbatch, 0)
        plsc.subcore_barrier()

        @pl.when(cid == c // 4)
        def _():
            pltpu.sync_copy(aggsh.at[pl.ds(sid * NROWT, NROWT), :],
                            agg_refs[c].at[pl.ds(sid * NROWT, NROWT), :])
        plsc.subcore_barrier()


def _agg(z_chunks, srcP, dstP, e_out, mxp):
    mesh = plsc.VectorSubcoreMesh(core_axis_name="c", subcore_axis_name="s")
    body = functools.partial(
        pl.kernel,
        out_type=[jax.ShapeDtypeStruct((NPAD, 128), jnp.float32)
                  for _ in range(NC)],
        mesh=mesh,
        scratch_types=[
            pltpu.VMEM((16, 128), jnp.float32),
            pltpu.VMEM((SB,), jnp.int32),
            pltpu.VMEM((SB,), jnp.int32),
            pltpu.VMEM((SB,), jnp.int32),
            pltpu.VMEM((SB,), jnp.float32),
            pltpu.VMEM((SB,), jnp.float32),
            pltpu.VMEM((SB,), jnp.float32),
            pltpu.VMEM((SB,), jnp.float32),
            pltpu.VMEM((SB,), jnp.float32),
            pltpu.VMEM((SB, 128), jnp.float32),
            pltpu.VMEM((SB, 128), jnp.float32),
            pltpu.VMEM((NF // 16,), jnp.float32),
            pltpu.VMEM((NF // 16,), jnp.float32),
            pltpu.VMEM_SHARED((NF,), jnp.float32),
            pltpu.VMEM_SHARED((NF,), jnp.float32),
            pltpu.VMEM_SHARED((NPAD, 128), jnp.float32),
            pltpu.SemaphoreType.DMA,
        ],
        compiler_params=pltpu.CompilerParams(needs_layout_passes=False),
    )

    def wrapped(*refs):
        z_refs = refs[:NC]
        src_ref, dst_ref, e_ref, mxp_ref = refs[NC:NC + 4]
        agg_refs = refs[NC + 4:NC + 4 + NC]
        scratch = refs[NC + 4 + NC:]
        _agg_body(z_refs, src_ref, dst_ref, e_ref, mxp_ref, agg_refs,
                  *scratch)

    return body(wrapped)(*z_chunks, srcP, dstP, e_out, mxp)


def _t1_body(h_ref, g_ref, b_ref, w_ref, hn_ref, *z_refs):
    hn = _ln(h_ref[...], g_ref[...], b_ref[...])
    hn_ref[...] = hn
    z = jnp.dot(hn, w_ref[...], preferred_element_type=jnp.float32)
    for c in range(NC):
        z_refs[c][...] = z[:, c * 128:(c + 1) * 128]


def _t1(h, g, b, Wg):
    outs = pl.pallas_call(
        _t1_body,
        grid=(NBLK,),
        in_specs=[
            pl.BlockSpec((RB, D), lambda i: (i, 0)),
            pl.BlockSpec((1, D), lambda i: (0, 0)),
            pl.BlockSpec((1, D), lambda i: (0, 0)),
            pl.BlockSpec((D, D), lambda i: (0, 0)),
        ],
        out_specs=[pl.BlockSpec((RB, D), lambda i: (i, 0))]
        + [pl.BlockSpec((RB, 128), lambda i: (i, 0)) for _ in range(NC)],
        out_shape=[jax.ShapeDtypeStruct((NPAD, D), jnp.float32)]
        + [jax.ShapeDtypeStruct((NPAD, 128), jnp.float32) for _ in range(NC)],
    )(h, g.reshape(1, D), b.reshape(1, D), Wg)
    return outs[0], outs[1:]


def _t2_body(*refs):
    agg_refs = refs[:NC]
    hn_ref, g_ref, b_ref, wd_ref, bd_ref, h_ref = refs[NC:]
    agg = jnp.concatenate([r[...] for r in agg_refs], axis=-1)
    hsum = agg + hn_ref[...]
    xr = _ln(hsum, g_ref[...], b_ref[...])
    d = jax.nn.gelu(jnp.dot(xr, wd_ref[...], preferred_element_type=jnp.float32)
                    + bd_ref[...])
    dr = d.reshape(RB, 4, D)
    h_ref[...] = dr[:, 0] + dr[:, 1] + dr[:, 2] + dr[:, 3] + xr


def _t2(agg_chunks, hn, g, b, Wd, bd):
    return pl.pallas_call(
        _t2_body,
        grid=(NBLK,),
        in_specs=[pl.BlockSpec((RB, 128), lambda i: (i, 0)) for _ in range(NC)]
        + [
            pl.BlockSpec((RB, D), lambda i: (i, 0)),
            pl.BlockSpec((1, D), lambda i: (0, 0)),
            pl.BlockSpec((1, D), lambda i: (0, 0)),
            pl.BlockSpec((D, D_FF), lambda i: (0, 0)),
            pl.BlockSpec((1, D_FF), lambda i: (0, 0)),
        ],
        out_specs=pl.BlockSpec((RB, D), lambda i: (i, 0)),
        out_shape=jax.ShapeDtypeStruct((NPAD, D), jnp.float32),
    )(*agg_chunks, hn, g.reshape(1, D), b.reshape(1, D), Wd,
      bd.reshape(1, D_FF))


def _tail_body(h_ref, wt_ref, bt_ref, o_ref):
    o_ref[...] = jnp.dot(h_ref[...], wt_ref[...],
                         preferred_element_type=jnp.float32) + bt_ref[...]


def _tail(h, Wt, bt):
    return pl.pallas_call(
        _tail_body,
        grid=(NBLK,),
        in_specs=[
            pl.BlockSpec((RB, D), lambda i: (i, 0)),
            pl.BlockSpec((D, OUT), lambda i: (0, 0)),
            pl.BlockSpec((1, OUT), lambda i: (0, 0)),
        ],
        out_specs=pl.BlockSpec((RB, OUT), lambda i: (i, 0)),
        out_shape=jax.ShapeDtypeStruct((NPAD, OUT), jnp.float32),
    )(h, Wt, bt.reshape(1, OUT))


def kernel(x, edge_index, W0, b0, ng_g, ng_b, Wg, ag, nd_g, nd_b, Wd, bd, Wt, bt):
    src = edge_index[0].astype(jnp.int32)
    dst = edge_index[1].astype(jnp.int32)
    srcP = jnp.pad(src, (0, EPAD - E))
    dstP = jnp.pad(dst, (0, EPAD - E), constant_values=N)
    xp = jnp.pad(x, ((0, NPAD - N), (0, 0)))
    h = _head(xp, W0, b0)
    for i in range(L):
        hn, z_chunks = _t1(h, ng_g[i], ng_b[i], Wg[i])
        a8 = ag[i].reshape(H, 2, 128).reshape(NC, 128)
        e_out, mxp = _edge_logits(z_chunks, a8, srcP, dstP)
        agg_chunks = _agg(z_chunks, srcP, dstP, e_out, mxp)
        h = _t2(agg_chunks, hn, nd_g[i], nd_b[i], Wd[i], bd[i])
    return _tail(h, Wt, bt)[:N]


# speedup vs baseline: 1.0064x; 1.0064x over previous
import functools

import jax
import jax.numpy as jnp
from jax import lax
from jax.experimental import pallas as pl
from jax.experimental.pallas import tpu as pltpu
from jax.experimental.pallas import tpu_sc as plsc

N = 10000
E = 160000
D_IN = 128
D = 1024
H = 4
DH = 256
D_FF = 4096
OUT = 256
L = 4

NPAD = 10240
RB = 256
NBLK = NPAD // RB

NTILE = 32
EPAD = 163840
ET = EPAD // NTILE
SB = 128
NSB = ET // SB
NC = 8
NF = NPAD * H
NQ = 4
NF4 = NF // NQ
NFS = NF4 // 16


def _shuf16(vec, perm):
    return lax.gather(
        vec, perm.reshape(16, 1),
        dimension_numbers=lax.GatherDimensionNumbers(
            offset_dims=(), collapsed_slice_dims=(0,), start_index_map=(0,)),
        slice_sizes=(1,),
        mode=lax.GatherScatterMode.PROMISE_IN_BOUNDS)


def _hsum16(vec):
    iota = lax.iota(jnp.int32, 16)
    for s in (8, 4, 2, 1):
        vec = vec + _shuf16(vec, iota ^ s)
    return vec


def _bc16(vec, i):
    idx = jnp.full((16, 1), i, jnp.int32)
    return lax.gather(
        vec, idx,
        dimension_numbers=lax.GatherDimensionNumbers(
            offset_dims=(), collapsed_slice_dims=(0,), start_index_map=(0,)),
        slice_sizes=(1,),
        mode=lax.GatherScatterMode.PROMISE_IN_BOUNDS)


_IOTA = None


SBA = 64
NSBA = ET // SBA


def _fire(z_refs, c0, c1, sidx, didx, bufs, sem):
    pltpu.async_copy(z_refs[c0].at[sidx], bufs[0], sem)
    pltpu.async_copy(z_refs[c1].at[sidx], bufs[1], sem)
    pltpu.async_copy(z_refs[c0].at[didx], bufs[2], sem)
    pltpu.async_copy(z_refs[c1].at[didx], bufs[3], sem)


def _drain(z_refs, bufs, sem):
    for b in bufs:
        pltpu.make_async_copy(z_refs[0], b, sem).wait()


def _edge_logits_body(z_refs, a_ref, src_ref, dst_ref, e_out, mxp_out,
                      a_v, b0, b1, b2, b3, b4, b5, b6, b7,
                      sidx0, didx0, sidx1, didx1, e_sb, mx, acc, tmp,
                      shared, sem0, sem1):
    cid = lax.axis_index("c")
    sid = lax.axis_index("s")
    wid = cid * 16 + sid
    base = wid * ET
    iota = lax.iota(jnp.int32, 16)
    lane15 = iota == 15
    hd = iota & 3
    msk = iota < 4
    sets = [[b0, b1, b2, b3], [b4, b5, b6, b7]]
    idxsets = [[sidx0, didx0], [sidx1, didx1]]
    sems = [sem0, sem1]

    pltpu.sync_copy(a_ref, a_v)

    def init_mx(v, _):
        mx[pl.ds(v * 16, 16)] = jnp.full((16,), -1e30, jnp.float32)
        return 0
    lax.fori_loop(0, NF // 16, init_mx, 0)

    a_all = [[a_v[c, pl.ds(dv * 16, 16)] for dv in range(8)]
             for c in range(NC)]

    def compute_hp(hp, bufs, didx):
        zsA, zsB, zdA, zdB = bufs
        aA = a_all[2 * hp]
        aB = a_all[2 * hp + 1]

        def edge(j, _):
            accv = jnp.zeros((16,), jnp.float32)
            for dv in range(8):
                sl = pl.ds(dv * 16, 16)
                t0 = zsA[j, sl] + zdA[j, sl]
                t1 = zsB[j, sl] + zdB[j, sl]
                accv += jnp.maximum(t0, t0 * 0.2) * aA[dv]
                accv += jnp.maximum(t1, t1 * 0.2) * aB[dv]
            tot = _hsum16(accv)
            plsc.store_scatter(
                e_sb, [jnp.full((16,), hp * SBA + j, jnp.int32)], tot,
                mask=lane15)
            return 0
        lax.fori_loop(0, SBA, edge, 0, unroll=2)

    def mx_pass(didx):
        def upd(j, _):
            jj = jnp.full((16,), j, jnp.int32)
            dstb = plsc.load_gather(didx, [jj])
            ev = plsc.load_gather(e_sb, [hd * SBA + jj], mask=msk)
            idxv = hd * NPAD + dstb
            cur = plsc.load_gather(mx, [idxv], mask=msk)
            plsc.store_scatter(mx, [idxv], jnp.maximum(cur, ev), mask=msk)
            return 0
        lax.fori_loop(0, SBA, upd, 0)

    def write_e(eb):
        for hh in range(H):
            pltpu.sync_copy(e_sb.at[pl.ds(hh * SBA, SBA)],
                            e_out.at[hh, pl.ds(eb, SBA)])

    pltpu.sync_copy(src_ref.at[pl.ds(base, SBA)], sidx0)
    pltpu.sync_copy(dst_ref.at[pl.ds(base, SBA)], didx0)
    _fire(z_refs, 0, 1, sidx0, didx0, sets[0], sems[0])

    def pair(k, _):
        for p in range(2):
            sb = k * 2 + p
            eb = base + sb * SBA
            s_idx, d_idx = idxsets[p]
            n_idx = idxsets[1 - p]
            for hp in range(H):
                if hp < 3:
                    _fire(z_refs, 2 * hp + 2, 2 * hp + 3, s_idx, d_idx,
                          sets[(hp + 1) & 1], sems[(hp + 1) & 1])
                else:
                    nb = eb + SBA if p == 0 else base + (k + 1) * 2 * SBA

                    @pl.when(nb < base + ET)
                    def _():
                        pltpu.sync_copy(src_ref.at[pl.ds(nb, SBA)], n_idx[0])
                        pltpu.sync_copy(dst_ref.at[pl.ds(nb, SBA)], n_idx[1])
                        _fire(z_refs, 0, 1, n_idx[0], n_idx[1], sets[0],
                              sems[0])
                _drain(z_refs, sets[hp & 1], sems[hp & 1])
                compute_hp(hp, sets[hp & 1], d_idx)
            write_e(eb)
            mx_pass(d_idx)
        return 0

    lax.fori_loop(0, NSBA // 2, pair, 0)

    for q in range(NQ):
        plsc.subcore_barrier()
        pltpu.sync_copy(mx.at[pl.ds(q * NF4, NF4)], shared.at[sid])
        plsc.subcore_barrier()
        off = sid * NFS
        pltpu.sync_copy(shared.at[0, pl.ds(off, NFS)], acc)
        for t in range(1, 16):
            pltpu.sync_copy(shared.at[t, pl.ds(off, NFS)], tmp)

            def mx_comb(v, _):
                sl = pl.ds(v * 16, 16)
                acc[sl] = jnp.maximum(acc[sl], tmp[sl])
                return 0
            lax.fori_loop(0, NFS // 16, mx_comb, 0)
        pltpu.sync_copy(acc, mxp_out.at[cid, pl.ds(q * NF4 + off, NFS)])


def _edge_logits(z_chunks, a8, srcP, dstP):
    mesh = plsc.VectorSubcoreMesh(core_axis_name="c", subcore_axis_name="s")
    body = functools.partial(
        pl.kernel,
        out_type=[
            jax.ShapeDtypeStruct((H, EPAD), jnp.float32),
            jax.ShapeDtypeStruct((2, NF), jnp.float32),
        ],
        mesh=mesh,
        scratch_types=[
            pltpu.VMEM((NC, 128), jnp.float32),
        ] + [pltpu.VMEM((SBA, 128), jnp.float32) for _ in range(8)] + [
            pltpu.VMEM((SBA,), jnp.int32),
            pltpu.VMEM((SBA,), jnp.int32),
            pltpu.VMEM((SBA,), jnp.int32),
            pltpu.VMEM((SBA,), jnp.int32),
            pltpu.VMEM((H * SBA,), jnp.float32),
            pltpu.VMEM((NF,), jnp.float32),
            pltpu.VMEM((NFS,), jnp.float32),
            pltpu.VMEM((NFS,), jnp.float32),
            pltpu.VMEM_SHARED((16, NF4), jnp.float32),
            pltpu.SemaphoreType.DMA,
            pltpu.SemaphoreType.DMA,
        ],
        compiler_params=pltpu.CompilerParams(needs_layout_passes=False),
    )

    def wrapped(*refs):
        z_refs = refs[:NC]
        rest = refs[NC:]
        _edge_logits_body(z_refs, *rest)

    return body(wrapped)(*z_chunks, a8, srcP, dstP)


def _ln(x, g, b):
    m = jnp.mean(x, axis=-1, keepdims=True)
    v = jnp.mean(jnp.square(x - m), axis=-1, keepdims=True)
    return (x - m) * jax.lax.rsqrt(v + 1e-3) * g + b


def _head_body(x_ref, w_ref, b_ref, o_ref):
    o_ref[...] = jnp.dot(x_ref[...], w_ref[...],
                         preferred_element_type=jnp.float32) + b_ref[...]


def _head(x, W0, b0):
    return pl.pallas_call(
        _head_body,
        grid=(NBLK,),
        in_specs=[
            pl.BlockSpec((RB, D_IN), lambda i: (i, 0)),
            pl.BlockSpec((D_IN, D), lambda i: (0, 0)),
            pl.BlockSpec((1, D), lambda i: (0, 0)),
        ],
        out_specs=pl.BlockSpec((RB, D), lambda i: (i, 0)),
        out_shape=jax.ShapeDtypeStruct((NPAD, D), jnp.float32),
    )(x, W0, b0.reshape(1, D))


EPT = EPAD // 16
NSB2 = EPT // SB
NROWT = NPAD // 16
NV = SB // 16


def _agg_body(z_refs, src_ref, dst_ref, e_ref, mxp_ref, agg_refs,
              zbuf, sidx, didx, didx2, ebuf, mxh, denh, exb, albuf,
              zrows, msgbuf, t0, t1, mxg, deng, aggsh, sem):
    cid = lax.axis_index("c")
    sid = lax.axis_index("s")

    def zero_zbuf(v, _):
        zbuf[v // 8, pl.ds((v % 8) * 16, 16)] = jnp.zeros((16,), jnp.float32)
        return 0
    lax.fori_loop(0, 128, zero_zbuf, 0)

    off = sid * (NF // 16)
    pltpu.sync_copy(mxp_ref.at[0, pl.ds(off, NF // 16)], t0)
    pltpu.sync_copy(mxp_ref.at[1, pl.ds(off, NF // 16)], t1)

    def mx_comb(v, _):
        sl = pl.ds(v * 16, 16)
        t0[sl] = jnp.maximum(t0[sl], t1[sl])
        return 0
    lax.fori_loop(0, NF // 256, mx_comb, 0)
    pltpu.sync_copy(t0, mxg.at[pl.ds(off, NF // 16)])

    def zt1(v, _):
        t1[pl.ds(v * 16, 16)] = jnp.zeros((16,), jnp.float32)
        return 0
    lax.fori_loop(0, NF // 256, zt1, 0)
    pltpu.sync_copy(t1, deng.at[pl.ds(off, NF // 16)])
    plsc.subcore_barrier()

    def sb_den(sb, _):
        eb = sid * EPT + sb * SB
        pltpu.sync_copy(dst_ref.at[pl.ds(eb, SB)], didx)
        for h in range(H):
            pltpu.sync_copy(e_ref.at[h, pl.ds(eb, SB)], ebuf)

            def mkidx(v, _):
                sl = pl.ds(v * 16, 16)
                didx2[sl] = didx[sl] + h * NPAD
                return 0
            lax.fori_loop(0, NV, mkidx, 0)
            pltpu.async_copy(mxg.at[didx2], mxh, sem).wait()

            def exv(v, _):
                sl = pl.ds(v * 16, 16)
                exb[sl] = jnp.exp(ebuf[sl] - mxh[sl])
                return 0
            lax.fori_loop(0, NV, exv, 0)
            pltpu.sync_copy(exb, deng.at[didx2], add=True)
        return 0
    lax.fori_loop(0, NSB2, sb_den, 0)
    plsc.subcore_barrier()

    for c in range(NC):
        @pl.when(cid == c // 4)
        def _():
            def zrow(r, _):
                pltpu.sync_copy(zbuf, aggsh.at[pl.ds(sid * NROWT + r * 16, 16), :])
                return 0
            lax.fori_loop(0, NROWT // 16, zrow, 0)
        plsc.subcore_barrier()

        @pl.when(cid == c // 4)
        def _():
            h = c >> 1

            def superbatch(sb, _):
                eb = sid * EPT + sb * SB
                pltpu.sync_copy(src_ref.at[pl.ds(eb, SB)], sidx)
                pltpu.sync_copy(dst_ref.at[pl.ds(eb, SB)], didx)
                pltpu.sync_copy(e_ref.at[h, pl.ds(eb, SB)], ebuf)

                def mkidx(v, _):
                    sl = pl.ds(v * 16, 16)
                    didx2[sl] = didx[sl] + h * NPAD
                    return 0
                lax.fori_loop(0, NV, mkidx, 0)
                pltpu.async_copy(mxg.at[didx2], mxh, sem).wait()
                pltpu.async_copy(deng.at[didx2], denh, sem).wait()
                pltpu.async_copy(z_refs[c].at[sidx], zrows, sem).wait()

                def alv(v, _):
                    sl = pl.ds(v * 16, 16)
                    albuf[sl] = jnp.exp(ebuf[sl] - mxh[sl]) / (denh[sl] + 1e-9)
                    return 0
                lax.fori_loop(0, NV, alv, 0)

                def edge(j, _):
                    jj = jnp.full((16,), j, jnp.int32)
                    albc = plsc.load_gather(albuf, [jj])
                    for dvv in range(8):
                        sl = pl.ds(dvv * 16, 16)
                        msgbuf[j, sl] = zrows[j, sl] * albc
                    return 0
                lax.fori_loop(0, SB, edge, 0, unroll=2)
                pltpu.sync_copy(msgbuf, aggsh.at[didx], add=True)
                return 0

            lax.fori_loop(0, NSB2, superbatch, 0)
        plsc.subcore_barrier()

        @pl.when(cid == c // 4)
        def _():
            pltpu.sync_copy(aggsh.at[pl.ds(sid * NROWT, NROWT), :],
                            agg_refs[c].at[pl.ds(sid * NROWT, NROWT), :])
        plsc.subcore_barrier()


def _agg(z_chunks, srcP, dstP, e_out, mxp):
    mesh = plsc.VectorSubcoreMesh(core_axis_name="c", subcore_axis_name="s")
    body = functools.partial(
        pl.kernel,
        out_type=[jax.ShapeDtypeStruct((NPAD, 128), jnp.float32)
                  for _ in range(NC)],
        mesh=mesh,
        scratch_types=[
            pltpu.VMEM((16, 128), jnp.float32),
            pltpu.VMEM((SB,), jnp.int32),
            pltpu.VMEM((SB,), jnp.int32),
            pltpu.VMEM((SB,), jnp.int32),
            pltpu.VMEM((SB,), jnp.float32),
            pltpu.VMEM((SB,), jnp.float32),
            pltpu.VMEM((SB,), jnp.float32),
            pltpu.VMEM((SB,), jnp.float32),
            pltpu.VMEM((SB,), jnp.float32),
            pltpu.VMEM((SB, 128), jnp.float32),
            pltpu.VMEM((SB, 128), jnp.float32),
            pltpu.VMEM((NF // 16,), jnp.float32),
            pltpu.VMEM((NF // 16,), jnp.float32),
            pltpu.VMEM_SHARED((NF,), jnp.float32),
            pltpu.VMEM_SHARED((NF,), jnp.float32),
            pltpu.VMEM_SHARED((NPAD, 128), jnp.float32),
            pltpu.SemaphoreType.DMA,
        ],
        compiler_params=pltpu.CompilerParams(needs_layout_passes=False),
    )

    def wrapped(*refs):
        z_refs = refs[:NC]
        src_ref, dst_ref, e_ref, mxp_ref = refs[NC:NC + 4]
        agg_refs = refs[NC + 4:NC + 4 + NC]
        scratch = refs[NC + 4 + NC:]
        _agg_body(z_refs, src_ref, dst_ref, e_ref, mxp_ref, agg_refs,
                  *scratch)

    return body(wrapped)(*z_chunks, srcP, dstP, e_out, mxp)


def _t1_body(h_ref, g_ref, b_ref, w_ref, hn_ref, *z_refs):
    hn = _ln(h_ref[...], g_ref[...], b_ref[...])
    hn_ref[...] = hn
    z = jnp.dot(hn, w_ref[...], preferred_element_type=jnp.float32)
    for c in range(NC):
        z_refs[c][...] = z[:, c * 128:(c + 1) * 128]


def _t1(h, g, b, Wg):
    outs = pl.pallas_call(
        _t1_body,
        grid=(NBLK,),
        in_specs=[
            pl.BlockSpec((RB, D), lambda i: (i, 0)),
            pl.BlockSpec((1, D), lambda i: (0, 0)),
            pl.BlockSpec((1, D), lambda i: (0, 0)),
            pl.BlockSpec((D, D), lambda i: (0, 0)),
        ],
        out_specs=[pl.BlockSpec((RB, D), lambda i: (i, 0))]
        + [pl.BlockSpec((RB, 128), lambda i: (i, 0)) for _ in range(NC)],
        out_shape=[jax.ShapeDtypeStruct((NPAD, D), jnp.float32)]
        + [jax.ShapeDtypeStruct((NPAD, 128), jnp.float32) for _ in range(NC)],
    )(h, g.reshape(1, D), b.reshape(1, D), Wg)
    return outs[0], outs[1:]


def _t2_body(*refs):
    agg_refs = refs[:NC]
    hn_ref, g_ref, b_ref, wd_ref, bd_ref, h_ref = refs[NC:]
    agg = jnp.concatenate([r[...] for r in agg_refs], axis=-1)
    hsum = agg + hn_ref[...]
    xr = _ln(hsum, g_ref[...], b_ref[...])
    d = jax.nn.gelu(jnp.dot(xr, wd_ref[...], preferred_element_type=jnp.float32)
                    + bd_ref[...])
    dr = d.reshape(RB, 4, D)
    h_ref[...] = dr[:, 0] + dr[:, 1] + dr[:, 2] + dr[:, 3] + xr


def _t2(agg_chunks, hn, g, b, Wd, bd):
    return pl.pallas_call(
        _t2_body,
        grid=(NBLK,),
        in_specs=[pl.BlockSpec((RB, 128), lambda i: (i, 0)) for _ in range(NC)]
        + [
            pl.BlockSpec((RB, D), lambda i: (i, 0)),
            pl.BlockSpec((1, D), lambda i: (0, 0)),
            pl.BlockSpec((1, D), lambda i: (0, 0)),
            pl.BlockSpec((D, D_FF), lambda i: (0, 0)),
            pl.BlockSpec((1, D_FF), lambda i: (0, 0)),
        ],
        out_specs=pl.BlockSpec((RB, D), lambda i: (i, 0)),
        out_shape=jax.ShapeDtypeStruct((NPAD, D), jnp.float32),
    )(*agg_chunks, hn, g.reshape(1, D), b.reshape(1, D), Wd,
      bd.reshape(1, D_FF))


def _tail_body(h_ref, wt_ref, bt_ref, o_ref):
    o_ref[...] = jnp.dot(h_ref[...], wt_ref[...],
                         preferred_element_type=jnp.float32) + bt_ref[...]


def _tail(h, Wt, bt):
    return pl.pallas_call(
        _tail_body,
        grid=(NBLK,),
        in_specs=[
            pl.BlockSpec((RB, D), lambda i: (i, 0)),
            pl.BlockSpec((D, OUT), lambda i: (0, 0)),
            pl.BlockSpec((1, OUT), lambda i: (0, 0)),
        ],
        out_specs=pl.BlockSpec((RB, OUT), lambda i: (i, 0)),
        out_shape=jax.ShapeDtypeStruct((NPAD, OUT), jnp.float32),
    )(h, Wt, bt.reshape(1, OUT))


def kernel(x, edge_index, W0, b0, ng_g, ng_b, Wg, ag, nd_g, nd_b, Wd, bd, Wt, bt):
    src = edge_index[0].astype(jnp.int32)
    dst = edge_index[1].astype(jnp.int32)
    srcP = jnp.pad(src, (0, EPAD - E))
    dstP = jnp.pad(dst, (0, EPAD - E), constant_values=N)
    xp = jnp.pad(x, ((0, NPAD - N), (0, 0)))
    h = _head(xp, W0, b0)
    for i in range(L):
        hn, z_chunks = _t1(h, ng_g[i], ng_b[i], Wg[i])
        a8 = ag[i].reshape(H, 2, 128).reshape(NC, 128)
        e_out, mxp = _edge_logits(z_chunks, a8, srcP, dstP)
        agg_chunks = _agg(z_chunks, srcP, dstP, e_out, mxp)
        h = _t2(agg_chunks, hn, nd_g[i], nd_b[i], Wd[i], bd[i])
    return _tail(h, Wt, bt)[:N]
